# baseline (speedup 1.0000x reference)
import jax
import jax.numpy as jnp
from jax import lax
from jax.experimental import pallas as pl
from jax.experimental.pallas import tpu as pltpu
from jax.experimental.pallas import tpu_sc as plsc

SPECIES_VALS = (1, 6, 7, 8, 15, 16)
NSP = 6
NBIN = 17
NA = 1048576
NS = 1024
NC, NSUBC, L = 2, 16, 16
NW = NC * NSUBC
CHUNK = NA // NW
SUB = 512
NSUB_W = CHUNK // SUB
NGRP = NSUB_W // L
QSIZE = CHUNK // NGRP
NSUB = NW * NSUB_W
OFF_W = NS // NW
GROWS = 48
CP = 49

_MESH = plsc.VectorSubcoreMesh(
    core_axis_name="c", subcore_axis_name="s", num_cores=NC, num_subcores=NSUBC)
_SC_PARAMS = pltpu.CompilerParams(needs_layout_passes=False)


def _wid():
  return lax.axis_index("s") * NC + lax.axis_index("c")


def _init_onehot_table(table_v):
  packed = {sv: 1 << (5 * s) for s, sv in enumerate(SPECIES_VALS)}
  for v in range(NBIN):
    table_v[pl.ds(v * L, L)] = jnp.full((L,), packed.get(v, 0), jnp.int32)


def _hist_body(species_hbm, cum_hbm, chunk_v, table_v, cum_v, sem):
  wid = _wid()
  base = wid * CHUNK
  descs = [
      pltpu.async_copy(species_hbm.at[pl.ds(base + q * QSIZE, QSIZE)],
                       chunk_v.at[pl.ds(q * QSIZE, QSIZE)], sem)
      for q in range(NGRP)
  ]

  lane = lax.iota(jnp.int32, L)
  zeros16 = jnp.zeros((L,), jnp.int32)
  _init_onehot_table(table_v)

  carry = [jnp.zeros((), jnp.int32) for _ in range(NSP)]
  for g in range(NGRP):
    descs[g].wait()
    qbase = g * QSIZE

    def sb_body(sbh, subcnts):
      for j in range(2):
        sb = sbh * 2 + j
        wides = [zeros16 for _ in range(NSP)]
        for half in range(2):
          acc = zeros16
          for k in range(L):
            v = chunk_v[pl.ds(qbase + sb * SUB + (half * L + k) * L, L)]
            acc = acc + plsc.load_gather(table_v, [v * L + lane])
          wides = [
              w + jnp.bitwise_and(lax.shift_right_logical(acc, 5 * s), 31)
              for s, w in enumerate(wides)]
        subcnts = tuple(
            jnp.where(lane == sb, jnp.broadcast_to(jnp.sum(w), (L,)), sc)
            for w, sc in zip(wides, subcnts))
      return subcnts

    subcnts = lax.fori_loop(0, L // 2, sb_body,
                            tuple(zeros16 for _ in range(NSP)))
    for s in range(NSP):
      inc = plsc.cumsum(subcnts[s]) + carry[s]
      plsc.store_scatter(cum_v, [(g * L + lane) * 8 + s], inc)
      carry[s] = inc[L - 1]

  pltpu.sync_copy(cum_v, cum_hbm.at[pl.ds(wid * NSUB_W * 8, NSUB_W * 8)])


def _compose_body(species2d_hbm, offsets_hbm, cum_hbm, out_hbm,
                  off_v, cum_v, cpref_v, idx_v, gbuf, table_v, pbuf_v, outbuf,
                  sem, sem2):
  wid = _wid()
  d_cum = pltpu.async_copy(cum_hbm, cum_v, sem2)
  pltpu.async_copy(offsets_hbm, off_v.at[pl.ds(0, NS)], sem).wait()

  lane = lax.iota(jnp.int32, L)
  sp_lane = jnp.minimum(lane, NSP - 1)
  zeros16 = jnp.zeros((L,), jnp.int32)

  for t in range(GROWS // L):
    ivec = wid * OFF_W + t * L + lane
    ovals = plsc.load_gather(off_v, [jnp.minimum(ivec, NS - 1)])
    idx_v[pl.ds(t * L, L)] = lax.shift_right_logical(ovals, 9)
  d_gbuf = pltpu.async_copy(species2d_hbm.at[idx_v], gbuf, sem)

  _init_onehot_table(table_v)

  d_cum.wait()
  for s in range(NSP):
    carry = jnp.zeros((), jnp.int32)
    for g in range(NW // L):
      tot = plsc.load_gather(
          cum_v, [((g * L + lane) * NSUB_W + (NSUB_W - 1)) * 8 + s])
      cpref_v[pl.ds(s * CP + g * L, L)] = plsc.cumsum(tot) - tot + carry
      carry = carry + jnp.sum(tot)
    cpref_v[pl.ds(s * CP + NW, L)] = jnp.broadcast_to(carry, (L,))

  d_gbuf.wait()

  def compute_p(i, r):
    icl = jnp.minimum(i, NS - 1)
    o = jnp.broadcast_to(off_v[pl.ds(icl, L)][0], (L,))
    o = jnp.where(i >= NS, NA, o)
    ocl = jnp.minimum(o, NA - 1)
    c = lax.shift_right_logical(ocl, 9)
    w = lax.shift_right_logical(c, 6)
    lsub = jnp.bitwise_and(c, NSUB_W - 1)
    rem = jnp.bitwise_and(ocl, SUB - 1)
    rvec = jnp.broadcast_to(r, (L,))

    accs = []
    for half in range(2):
      acc = zeros16
      for k in range(L):
        kk = half * L + k
        v = plsc.load_gather(gbuf, [rvec, kk * L + lane])
        m = (kk * L + lane) < rem
        oh = plsc.load_gather(table_v, [v * L + lane])
        acc = acc + jnp.where(m, oh, 0)
      accs.append(acc)

    fm = 0x01F07C1F
    ev = jnp.sum(jnp.bitwise_and(accs[0], fm) + jnp.bitwise_and(accs[1], fm))
    od = jnp.sum(jnp.bitwise_and(lax.shift_right_logical(accs[0], 5), fm) +
                 jnp.bitwise_and(lax.shift_right_logical(accs[1], 5), fm))
    partial = jnp.zeros((L,), jnp.int32)
    for s in range(NSP):
      src = ev if s % 2 == 0 else od
      cnt = jnp.bitwise_and(lax.shift_right_logical(src, 10 * (s // 2)), 1023)
      partial = jnp.where(lane == s, jnp.broadcast_to(cnt, (L,)), partial)

    cm1 = jnp.maximum(c, 1)[0] - 1
    base = plsc.load_gather(cpref_v, [sp_lane * CP + w], mask=lane < NSP)
    local = jnp.where(lsub > 0, cum_v[pl.ds(cm1 * 8, L)], 0)
    total = plsc.load_gather(cpref_v, [sp_lane * CP + NW], mask=lane < NSP)
    return jnp.where(i >= NS, total, base + local + partial)

  def _p_pass(rt, acc):
    for j in range(3):
      r = rt * 3 + j
      pbuf_v[pl.ds(r * L, L)] = compute_p(wid * OFF_W + r, r)
    return acc

  lax.fori_loop(0, (OFF_W + 1) // 3, _p_pass, jnp.zeros((), jnp.int32))

  for r in range(1, OFF_W + 1):
    vals = (pbuf_v[pl.ds(r * L, L)] -
            pbuf_v[pl.ds((r - 1) * L, L)]).astype(jnp.float32)
    bpos = (r - 1) * NSP
    plsc.store_scatter(outbuf, [jnp.minimum(bpos + lane, OFF_W * NSP - 1)],
                       vals, mask=lane < NSP)

  pltpu.sync_copy(outbuf, out_hbm.at[pl.ds(wid * OFF_W * NSP, OFF_W * NSP)])


_hist_call = pl.kernel(
    _hist_body,
    out_type=jax.ShapeDtypeStruct((NSUB * 8,), jnp.int32),
    mesh=_MESH,
    compiler_params=_SC_PARAMS,
    scratch_types=[
        pltpu.VMEM((CHUNK,), jnp.int32),
        pltpu.VMEM((NBIN * L,), jnp.int32),
        pltpu.VMEM((NSUB_W * 8,), jnp.int32),
        pltpu.SemaphoreType.DMA,
    ],
)

_compose_call = pl.kernel(
    _compose_body,
    out_type=jax.ShapeDtypeStruct((NS * NSP,), jnp.float32),
    mesh=_MESH,
    compiler_params=_SC_PARAMS,
    scratch_types=[
        pltpu.VMEM((NS + L,), jnp.int32),
        pltpu.VMEM((NSUB * 8,), jnp.int32),
        pltpu.VMEM((NSP * CP,), jnp.int32),
        pltpu.VMEM((GROWS,), jnp.int32),
        pltpu.VMEM((GROWS, SUB), jnp.int32),
        pltpu.VMEM((NBIN * L,), jnp.int32),
        pltpu.VMEM(((OFF_W + 1) * L,), jnp.int32),
        pltpu.VMEM((OFF_W * NSP,), jnp.float32),
        pltpu.SemaphoreType.DMA,
        pltpu.SemaphoreType.DMA,
    ],
)


@jax.jit
def kernel(positions, cells, species, cell_shifts, centers, pairs,
           structure_centers, structure_pairs, structure_offsets):
  cum = _hist_call(species)
  flat = _compose_call(species.reshape(NSUB, SUB), structure_offsets, cum)
  return flat.reshape(NS, NSP)


# speedup vs baseline: 1.0362x; 1.0362x over previous
import jax
import jax.numpy as jnp
from jax import lax
from jax.experimental import pallas as pl
from jax.experimental.pallas import tpu as pltpu
from jax.experimental.pallas import tpu_sc as plsc

SPECIES_VALS = (1, 6, 7, 8, 15, 16)
NSP = 6
NBIN = 17
NA = 1048576
NS = 1024
NC, NSUBC, L = 2, 16, 16
NW = NC * NSUBC
CHUNK = NA // NW
SUB = 512
NSUB_W = CHUNK // SUB
NGRP = NSUB_W // L
QSIZE = CHUNK // NGRP
NSUB = NW * NSUB_W
OFF_W = NS // NW
GROWS = 48
CP = 49

_MESH = plsc.VectorSubcoreMesh(
    core_axis_name="c", subcore_axis_name="s", num_cores=NC, num_subcores=NSUBC)
_SC_PARAMS = pltpu.CompilerParams(needs_layout_passes=False)


def _wid():
  return lax.axis_index("s") * NC + lax.axis_index("c")


def _init_onehot_table(table_v):
  packed = {sv: 1 << (5 * s) for s, sv in enumerate(SPECIES_VALS)}
  for v in range(NBIN):
    table_v[pl.ds(v * L, L)] = jnp.full((L,), packed.get(v, 0), jnp.int32)


def _hist_body(species_hbm, cum_hbm, chunk_v, table_v, cum_v, sem):
  wid = _wid()
  base = wid * CHUNK
  descs = [
      pltpu.async_copy(species_hbm.at[pl.ds(base + q * QSIZE, QSIZE)],
                       chunk_v.at[pl.ds(q * QSIZE, QSIZE)], sem)
      for q in range(NGRP)
  ]

  lane = lax.iota(jnp.int32, L)
  zeros16 = jnp.zeros((L,), jnp.int32)
  _init_onehot_table(table_v)

  carry = [jnp.zeros((), jnp.int32) for _ in range(NSP)]
  for g in range(NGRP):
    descs[g].wait()
    qbase = g * QSIZE

    def sb_body(sb, subcnts):
      wides = [zeros16 for _ in range(NSP)]
      for half in range(2):
        acc = zeros16
        for k in range(L):
          v = chunk_v[pl.ds(qbase + sb * SUB + (half * L + k) * L, L)]
          acc = acc + plsc.load_gather(table_v, [v * L + lane])
        wides = [
            w + jnp.bitwise_and(lax.shift_right_logical(acc, 5 * s), 31)
            for s, w in enumerate(wides)]
      return tuple(
          jnp.where(lane == sb, jnp.broadcast_to(jnp.sum(w), (L,)), sc)
          for w, sc in zip(wides, subcnts))

    subcnts = lax.fori_loop(0, L, sb_body, tuple(zeros16 for _ in range(NSP)))
    for s in range(NSP):
      inc = plsc.cumsum(subcnts[s]) + carry[s]
      plsc.store_scatter(cum_v, [(g * L + lane) * 8 + s], inc)
      carry[s] = inc[L - 1]

  pltpu.sync_copy(cum_v, cum_hbm.at[pl.ds(wid * NSUB_W * 8, NSUB_W * 8)])


def _compose_body(species2d_hbm, offsets_hbm, cum_hbm, out_hbm,
                  off_v, cum_v, cpref_v, idx_v, gbuf, table_v, pbuf_v, outbuf,
                  sem, sem2):
  wid = _wid()
  d_cum = pltpu.async_copy(cum_hbm, cum_v, sem2)
  pltpu.async_copy(offsets_hbm, off_v.at[pl.ds(0, NS)], sem).wait()

  lane = lax.iota(jnp.int32, L)
  sp_lane = jnp.minimum(lane, NSP - 1)
  zeros16 = jnp.zeros((L,), jnp.int32)

  for t in range(GROWS // L):
    ivec = wid * OFF_W + t * L + lane
    ovals = plsc.load_gather(off_v, [jnp.minimum(ivec, NS - 1)])
    idx_v[pl.ds(t * L, L)] = lax.shift_right_logical(ovals, 9)
  d_gbuf = pltpu.async_copy(species2d_hbm.at[idx_v], gbuf, sem)

  _init_onehot_table(table_v)

  d_cum.wait()
  for s in range(NSP):
    carry = jnp.zeros((), jnp.int32)
    for g in range(NW // L):
      tot = plsc.load_gather(
          cum_v, [((g * L + lane) * NSUB_W + (NSUB_W - 1)) * 8 + s])
      cpref_v[pl.ds(s * CP + g * L, L)] = plsc.cumsum(tot) - tot + carry
      carry = carry + jnp.sum(tot)
    cpref_v[pl.ds(s * CP + NW, L)] = jnp.broadcast_to(carry, (L,))

  d_gbuf.wait()

  def compute_p(i, r):
    icl = jnp.minimum(i, NS - 1)
    o = jnp.broadcast_to(off_v[pl.ds(icl, L)][0], (L,))
    o = jnp.where(i >= NS, NA, o)
    ocl = jnp.minimum(o, NA - 1)
    c = lax.shift_right_logical(ocl, 9)
    w = lax.shift_right_logical(c, 6)
    lsub = jnp.bitwise_and(c, NSUB_W - 1)
    rem = jnp.bitwise_and(ocl, SUB - 1)
    rvec = jnp.broadcast_to(r, (L,))

    accs = []
    for half in range(2):
      acc = zeros16
      for k in range(L):
        kk = half * L + k
        v = plsc.load_gather(gbuf, [rvec, kk * L + lane])
        m = (kk * L + lane) < rem
        oh = plsc.load_gather(table_v, [v * L + lane])
        acc = acc + jnp.where(m, oh, 0)
      accs.append(acc)

    fm = 0x01F07C1F
    ev = jnp.sum(jnp.bitwise_and(accs[0], fm) + jnp.bitwise_and(accs[1], fm))
    od = jnp.sum(jnp.bitwise_and(lax.shift_right_logical(accs[0], 5), fm) +
                 jnp.bitwise_and(lax.shift_right_logical(accs[1], 5), fm))
    partial = jnp.zeros((L,), jnp.int32)
    for s in range(NSP):
      src = ev if s % 2 == 0 else od
      cnt = jnp.bitwise_and(lax.shift_right_logical(src, 10 * (s // 2)), 1023)
      partial = jnp.where(lane == s, jnp.broadcast_to(cnt, (L,)), partial)

    cm1 = jnp.maximum(c, 1)[0] - 1
    base = plsc.load_gather(cpref_v, [sp_lane * CP + w], mask=lane < NSP)
    local = jnp.where(lsub > 0, cum_v[pl.ds(cm1 * 8, L)], 0)
    total = plsc.load_gather(cpref_v, [sp_lane * CP + NW], mask=lane < NSP)
    return jnp.where(i >= NS, total, base + local + partial)

  def _p_pass(rt, acc):
    for j in range(6):
      r = jnp.minimum(rt * 6 + j, OFF_W)
      pbuf_v[pl.ds(r * L, L)] = compute_p(wid * OFF_W + r, r)
    return acc

  lax.fori_loop(0, (OFF_W + 1 + 5) // 6, _p_pass, jnp.zeros((), jnp.int32))

  for r in range(1, OFF_W + 1):
    vals = (pbuf_v[pl.ds(r * L, L)] -
            pbuf_v[pl.ds((r - 1) * L, L)]).astype(jnp.float32)
    bpos = (r - 1) * NSP
    plsc.store_scatter(outbuf, [jnp.minimum(bpos + lane, OFF_W * NSP - 1)],
                       vals, mask=lane < NSP)

  pltpu.sync_copy(outbuf, out_hbm.at[pl.ds(wid * OFF_W * NSP, OFF_W * NSP)])


_hist_call = pl.kernel(
    _hist_body,
    out_type=jax.ShapeDtypeStruct((NSUB * 8,), jnp.int32),
    mesh=_MESH,
    compiler_params=_SC_PARAMS,
    scratch_types=[
        pltpu.VMEM((CHUNK,), jnp.int32),
        pltpu.VMEM((NBIN * L,), jnp.int32),
        pltpu.VMEM((NSUB_W * 8,), jnp.int32),
        pltpu.SemaphoreType.DMA,
    ],
)

_compose_call = pl.kernel(
    _compose_body,
    out_type=jax.ShapeDtypeStruct((NS * NSP,), jnp.float32),
    mesh=_MESH,
    compiler_params=_SC_PARAMS,
    scratch_types=[
        pltpu.VMEM((NS + L,), jnp.int32),
        pltpu.VMEM((NSUB * 8,), jnp.int32),
        pltpu.VMEM((NSP * CP,), jnp.int32),
        pltpu.VMEM((GROWS,), jnp.int32),
        pltpu.VMEM((GROWS, SUB), jnp.int32),
        pltpu.VMEM((NBIN * L,), jnp.int32),
        pltpu.VMEM(((OFF_W + 1) * L,), jnp.int32),
        pltpu.VMEM((OFF_W * NSP,), jnp.float32),
        pltpu.SemaphoreType.DMA,
        pltpu.SemaphoreType.DMA,
    ],
)


@jax.jit
def kernel(positions, cells, species, cell_shifts, centers, pairs,
           structure_centers, structure_pairs, structure_offsets):
  cum = _hist_call(species)
  flat = _compose_call(species.reshape(NSUB, SUB), structure_offsets, cum)
  return flat.reshape(NS, NSP)


# speedup vs baseline: 1.0582x; 1.0213x over previous
import jax
import jax.numpy as jnp
from jax import lax
from jax.experimental import pallas as pl
from jax.experimental.pallas import tpu as pltpu
from jax.experimental.pallas import tpu_sc as plsc

SPECIES_VALS = (1, 6, 7, 8, 15, 16)
NSP = 6
NBIN = 17
NA = 1048576
NS = 1024
NC, NSUBC, L = 2, 16, 16
NW = NC * NSUBC
CHUNK = NA // NW
SUB = 512
NSUB_W = CHUNK // SUB
NGRP = NSUB_W // L
QSIZE = CHUNK // NGRP
NSUB = NW * NSUB_W
OFF_W = NS // NW
GROWS = 48
CP = 49

_MESH = plsc.VectorSubcoreMesh(
    core_axis_name="c", subcore_axis_name="s", num_cores=NC, num_subcores=NSUBC)
_SC_PARAMS = pltpu.CompilerParams(needs_layout_passes=False)


def _wid():
  return lax.axis_index("s") * NC + lax.axis_index("c")


def _init_onehot_table(table_v):
  packed = {sv: 1 << (5 * s) for s, sv in enumerate(SPECIES_VALS)}
  for v in range(NBIN):
    table_v[pl.ds(v * L, L)] = jnp.full((L,), packed.get(v, 0), jnp.int32)


def _hist_body(species_hbm, cum_hbm, chunk_v, table_v, cum_v, sem):
  wid = _wid()
  base = wid * CHUNK
  descs = [
      pltpu.async_copy(species_hbm.at[pl.ds(base + q * QSIZE, QSIZE)],
                       chunk_v.at[pl.ds(q * QSIZE, QSIZE)], sem)
      for q in range(NGRP)
  ]

  lane = lax.iota(jnp.int32, L)
  zeros16 = jnp.zeros((L,), jnp.int32)
  _init_onehot_table(table_v)

  carry = [jnp.zeros((), jnp.int32) for _ in range(NSP)]
  for g in range(NGRP):
    descs[g].wait()
    qbase = g * QSIZE

    def sb_body(sb, subcnts):
      wides = [zeros16 for _ in range(NSP)]
      for half in range(2):
        acc = zeros16
        for k in range(L):
          v = chunk_v[pl.ds(qbase + sb * SUB + (half * L + k) * L, L)]
          acc = acc + plsc.load_gather(table_v, [v * L + lane])
        wides = [
            w + jnp.bitwise_and(lax.shift_right_logical(acc, 5 * s), 31)
            for s, w in enumerate(wides)]
      return tuple(
          jnp.where(lane == sb, jnp.broadcast_to(jnp.sum(w), (L,)), sc)
          for w, sc in zip(wides, subcnts))

    subcnts = lax.fori_loop(0, L, sb_body, tuple(zeros16 for _ in range(NSP)))
    for s in range(NSP):
      inc = plsc.cumsum(subcnts[s]) + carry[s]
      plsc.store_scatter(cum_v, [(g * L + lane) * 8 + s], inc)
      carry[s] = inc[L - 1]

  pltpu.sync_copy(cum_v, cum_hbm.at[pl.ds(wid * NSUB_W * 8, NSUB_W * 8)])


def _compose_body(species2d_hbm, offsets_hbm, cum_hbm, out_hbm,
                  off_v, cum_v, cpref_v, idx_v, gbuf, table_v, pbuf_v, outbuf,
                  sem, sem2):
  wid = _wid()
  d_cum = pltpu.async_copy(cum_hbm, cum_v, sem2)
  pltpu.async_copy(offsets_hbm, off_v.at[pl.ds(0, NS)], sem).wait()

  lane = lax.iota(jnp.int32, L)
  sp_lane = jnp.minimum(lane, NSP - 1)
  zeros16 = jnp.zeros((L,), jnp.int32)

  for t in range(GROWS // L):
    ivec = wid * OFF_W + t * L + lane
    ovals = plsc.load_gather(off_v, [jnp.minimum(ivec, NS - 1)])
    idx_v[pl.ds(t * L, L)] = lax.shift_right_logical(ovals, 9)
  d_gbuf = pltpu.async_copy(species2d_hbm.at[idx_v], gbuf, sem)

  _init_onehot_table(table_v)

  d_cum.wait()
  for s in range(NSP):
    carry = jnp.zeros((), jnp.int32)
    for g in range(NW // L):
      tot = plsc.load_gather(
          cum_v, [((g * L + lane) * NSUB_W + (NSUB_W - 1)) * 8 + s])
      cpref_v[pl.ds(s * CP + g * L, L)] = plsc.cumsum(tot) - tot + carry
      carry = carry + jnp.sum(tot)
    cpref_v[pl.ds(s * CP + NW, L)] = jnp.broadcast_to(carry, (L,))

  d_gbuf.wait()

  def compute_p(i, r):
    icl = jnp.minimum(i, NS - 1)
    o = jnp.broadcast_to(off_v[pl.ds(icl, L)][0], (L,))
    o = jnp.where(i >= NS, NA, o)
    ocl = jnp.minimum(o, NA - 1)
    c = lax.shift_right_logical(ocl, 9)
    w = lax.shift_right_logical(c, 6)
    lsub = jnp.bitwise_and(c, NSUB_W - 1)
    rem = jnp.bitwise_and(ocl, SUB - 1)
    rvec = jnp.broadcast_to(r, (L,))

    accs = []
    for half in range(2):
      acc = zeros16
      for k in range(L):
        kk = half * L + k
        v = plsc.load_gather(gbuf, [rvec, kk * L + lane])
        m = (kk * L + lane) < rem
        oh = plsc.load_gather(table_v, [v * L + lane])
        acc = acc + jnp.where(m, oh, 0)
      accs.append(acc)

    fm = 0x01F07C1F
    ev = jnp.sum(jnp.bitwise_and(accs[0], fm) + jnp.bitwise_and(accs[1], fm))
    od = jnp.sum(jnp.bitwise_and(lax.shift_right_logical(accs[0], 5), fm) +
                 jnp.bitwise_and(lax.shift_right_logical(accs[1], 5), fm))
    partial = jnp.zeros((L,), jnp.int32)
    for s in range(NSP):
      src = ev if s % 2 == 0 else od
      cnt = jnp.bitwise_and(lax.shift_right_logical(src, 10 * (s // 2)), 1023)
      partial = jnp.where(lane == s, jnp.broadcast_to(cnt, (L,)), partial)

    cm1 = jnp.maximum(c, 1)[0] - 1
    base = plsc.load_gather(cpref_v, [sp_lane * CP + w], mask=lane < NSP)
    local = jnp.where(lsub > 0, cum_v[pl.ds(cm1 * 8, L)], 0)
    total = plsc.load_gather(cpref_v, [sp_lane * CP + NW], mask=lane < NSP)
    return jnp.where(i >= NS, total, base + local + partial)

  def _p_pass(rt, acc):
    for j in range(3):
      r = rt * 3 + j
      pbuf_v[pl.ds(r * L, L)] = compute_p(wid * OFF_W + r, r)
    return acc

  lax.fori_loop(0, (OFF_W + 1) // 3, _p_pass, jnp.zeros((), jnp.int32))

  for r in range(1, OFF_W + 1):
    vals = (pbuf_v[pl.ds(r * L, L)] -
            pbuf_v[pl.ds((r - 1) * L, L)]).astype(jnp.float32)
    bpos = (r - 1) * NSP
    plsc.store_scatter(outbuf, [jnp.minimum(bpos + lane, OFF_W * NSP - 1)],
                       vals, mask=lane < NSP)

  pltpu.sync_copy(outbuf, out_hbm.at[pl.ds(wid * OFF_W * NSP, OFF_W * NSP)])


_hist_call = pl.kernel(
    _hist_body,
    out_type=jax.ShapeDtypeStruct((NSUB * 8,), jnp.int32),
    mesh=_MESH,
    compiler_params=_SC_PARAMS,
    scratch_types=[
        pltpu.VMEM((CHUNK,), jnp.int32),
        pltpu.VMEM((NBIN * L,), jnp.int32),
        pltpu.VMEM((NSUB_W * 8,), jnp.int32),
        pltpu.SemaphoreType.DMA,
    ],
)

_compose_call = pl.kernel(
    _compose_body,
    out_type=jax.ShapeDtypeStruct((NS * NSP,), jnp.float32),
    mesh=_MESH,
    compiler_params=_SC_PARAMS,
    scratch_types=[
        pltpu.VMEM((NS + L,), jnp.int32),
        pltpu.VMEM((NSUB * 8,), jnp.int32),
        pltpu.VMEM((NSP * CP,), jnp.int32),
        pltpu.VMEM((GROWS,), jnp.int32),
        pltpu.VMEM((GROWS, SUB), jnp.int32),
        pltpu.VMEM((NBIN * L,), jnp.int32),
        pltpu.VMEM(((OFF_W + 1) * L,), jnp.int32),
        pltpu.VMEM((OFF_W * NSP,), jnp.float32),
        pltpu.SemaphoreType.DMA,
        pltpu.SemaphoreType.DMA,
    ],
)


@jax.jit
def kernel(positions, cells, species, cell_shifts, centers, pairs,
           structure_centers, structure_pairs, structure_offsets):
  cum = _hist_call(species)
  flat = _compose_call(species.reshape(NSUB, SUB), structure_offsets, cum)
  return flat.reshape(NS, NSP)
